# baseline (speedup 1.0000x reference)
import math

import jax
import jax.numpy as jnp
from jax.experimental import pallas as pl

E = 8
C_IN = 96
HID = 96
C_OUT = 96
R_IN = 8

ROWS = 32
_SQRT1_2 = 1.0 / math.sqrt(2.0)


def _moe_body(x_ref, r_ref, w1_ref, b1_ref, w2_ref, b2_ref, w3_ref, b3_ref,
              wr_ref, br_ref, o_ref):
    hb, w = x_ref.shape[2], x_ref.shape[3]
    t = hb * w
    xt = x_ref[0].reshape(C_IN, t)
    rt = r_ref[0].reshape(R_IN, t)

    logits = jnp.dot(wr_ref[...], rt, preferred_element_type=jnp.float32)
    logits = logits + br_ref[...].reshape(E, 1)

    eidx = jax.lax.broadcasted_iota(jnp.int32, (E, t), 0)
    v1 = jnp.max(logits, axis=0, keepdims=True)
    idx1 = jnp.min(jnp.where(logits == v1, eidx, E), axis=0, keepdims=True)
    sel1 = eidx == idx1
    rest = jnp.where(sel1, -jnp.inf, logits)
    v2 = jnp.max(rest, axis=0, keepdims=True)
    idx2 = jnp.min(jnp.where(rest == v2, eidx, E), axis=0, keepdims=True)
    sel2 = eidx == idx2
    g1 = 1.0 / (1.0 + jnp.exp(v2 - v1))
    g2 = 1.0 - g1
    gates = jnp.where(sel1, g1, 0.0) + jnp.where(sel2, g2, 0.0)

    h = jnp.dot(w1_ref[...], xt, preferred_element_type=jnp.float32)
    h = h + b1_ref[...]
    g = h * (1.0 + jax.lax.erf(h))

    acc = jnp.dot(b3_ref[...], gates, preferred_element_type=jnp.float32)
    for p in range(E // 2):
        h2 = jnp.dot(w2_ref[p], g[192 * p:192 * (p + 1)],
                     preferred_element_type=jnp.float32)
        h2 = h2 + b2_ref[p]
        g2v = h2 * (1.0 + jax.lax.erf(h2))
        gb = jnp.concatenate(
            [jnp.broadcast_to(gates[2 * p:2 * p + 1], (HID, t)),
             jnp.broadcast_to(gates[2 * p + 1:2 * p + 2], (HID, t))], axis=0)
        acc = acc + jnp.dot(w3_ref[p], g2v * gb,
                            preferred_element_type=jnp.float32)
    o_ref[0] = acc.reshape(C_OUT, hb, w)


def kernel(x, router_input, W1, b1, W2, b2, W3, b3, Wr, br):
    B, _, H, W = x.shape
    hb = ROWS
    nt = H // hb

    w1s = (W1 * _SQRT1_2).reshape(E * HID, C_IN)
    b1s = (b1 * _SQRT1_2).reshape(E * HID, 1)
    w2h = W2 * 0.5
    w2p = jnp.zeros((E // 2, 2 * HID, 2 * HID), jnp.float32)
    w2p = w2p.at[:, :HID, :HID].set(w2h[0::2])
    w2p = w2p.at[:, HID:, HID:].set(w2h[1::2])
    b2p = (b2 * _SQRT1_2).reshape(E // 2, 2 * HID, 1)
    w3h = W3 * _SQRT1_2
    w3p = jnp.concatenate([w3h[0::2], w3h[1::2]], axis=2)
    b3t = b3.T

    grid = (B, nt)
    out = pl.pallas_call(
        _moe_body,
        grid=grid,
        in_specs=[
            pl.BlockSpec((1, C_IN, hb, W), lambda b, i: (b, 0, i, 0)),
            pl.BlockSpec((1, R_IN, hb, W), lambda b, i: (b, 0, i, 0)),
            pl.BlockSpec((E * HID, C_IN), lambda b, i: (0, 0)),
            pl.BlockSpec((E * HID, 1), lambda b, i: (0, 0)),
            pl.BlockSpec((E // 2, 2 * HID, 2 * HID), lambda b, i: (0, 0, 0)),
            pl.BlockSpec((E // 2, 2 * HID, 1), lambda b, i: (0, 0, 0)),
            pl.BlockSpec((E // 2, C_OUT, 2 * HID), lambda b, i: (0, 0, 0)),
            pl.BlockSpec((C_OUT, E), lambda b, i: (0, 0)),
            pl.BlockSpec((E, R_IN), lambda b, i: (0, 0)),
            pl.BlockSpec((1, E), lambda b, i: (0, 0)),
        ],
        out_specs=pl.BlockSpec((1, C_OUT, hb, W), lambda b, i: (b, 0, i, 0)),
        out_shape=jax.ShapeDtypeStruct((B, C_OUT, H, W), jnp.float32),
    )(x, router_input, w1s, b1s, w2p, b2p, w3p, b3t, Wr, br.reshape(1, E))
    return out


# speedup vs baseline: 1.0124x; 1.0124x over previous
import math

import jax
import jax.numpy as jnp
from jax.experimental import pallas as pl

E = 8
C_IN = 96
HID = 96
C_OUT = 96
R_IN = 8

ROWS = 16
_SQRT1_2 = 1.0 / math.sqrt(2.0)


def _moe_body(x_ref, r_ref, w1_ref, b1_ref, w2_ref, b2_ref, w3_ref, b3_ref,
              wr_ref, br_ref, o_ref):
    hb, w = x_ref.shape[2], x_ref.shape[3]
    t = hb * w
    xt = x_ref[0].reshape(C_IN, t)
    rt = r_ref[0].reshape(R_IN, t)

    logits = jnp.dot(wr_ref[...], rt, preferred_element_type=jnp.float32)
    logits = logits + br_ref[...].reshape(E, 1)

    eidx = jax.lax.broadcasted_iota(jnp.int32, (E, t), 0)
    v1 = jnp.max(logits, axis=0, keepdims=True)
    idx1 = jnp.min(jnp.where(logits == v1, eidx, E), axis=0, keepdims=True)
    sel1 = eidx == idx1
    rest = jnp.where(sel1, -jnp.inf, logits)
    v2 = jnp.max(rest, axis=0, keepdims=True)
    idx2 = jnp.min(jnp.where(rest == v2, eidx, E), axis=0, keepdims=True)
    sel2 = eidx == idx2
    g1 = 1.0 / (1.0 + jnp.exp(v2 - v1))
    g2 = 1.0 - g1
    gates = jnp.where(sel1, g1, 0.0) + jnp.where(sel2, g2, 0.0)

    h = jnp.dot(w1_ref[...], xt, preferred_element_type=jnp.float32)
    h = h + b1_ref[...]
    g = h * (1.0 + jax.lax.erf(h))

    acc = jnp.dot(b3_ref[...], gates, preferred_element_type=jnp.float32)
    for p in range(E // 2):
        h2 = jnp.dot(w2_ref[p], g[192 * p:192 * (p + 1)],
                     preferred_element_type=jnp.float32)
        h2 = h2 + b2_ref[p]
        g2v = h2 * (1.0 + jax.lax.erf(h2))
        gb = jnp.concatenate(
            [jnp.broadcast_to(gates[2 * p:2 * p + 1], (HID, t)),
             jnp.broadcast_to(gates[2 * p + 1:2 * p + 2], (HID, t))], axis=0)
        acc = acc + jnp.dot(w3_ref[p], g2v * gb,
                            preferred_element_type=jnp.float32)
    o_ref[0] = acc.reshape(C_OUT, hb, w)


def kernel(x, router_input, W1, b1, W2, b2, W3, b3, Wr, br):
    B, _, H, W = x.shape
    hb = ROWS
    nt = H // hb

    w1s = (W1 * _SQRT1_2).reshape(E * HID, C_IN)
    b1s = (b1 * _SQRT1_2).reshape(E * HID, 1)
    w2h = W2 * 0.5
    w2p = jnp.zeros((E // 2, 2 * HID, 2 * HID), jnp.float32)
    w2p = w2p.at[:, :HID, :HID].set(w2h[0::2])
    w2p = w2p.at[:, HID:, HID:].set(w2h[1::2])
    b2p = (b2 * _SQRT1_2).reshape(E // 2, 2 * HID, 1)
    w3h = W3 * _SQRT1_2
    w3p = jnp.concatenate([w3h[0::2], w3h[1::2]], axis=2)
    b3t = b3.T

    grid = (B, nt)
    out = pl.pallas_call(
        _moe_body,
        grid=grid,
        in_specs=[
            pl.BlockSpec((1, C_IN, hb, W), lambda b, i: (b, 0, i, 0)),
            pl.BlockSpec((1, R_IN, hb, W), lambda b, i: (b, 0, i, 0)),
            pl.BlockSpec((E * HID, C_IN), lambda b, i: (0, 0)),
            pl.BlockSpec((E * HID, 1), lambda b, i: (0, 0)),
            pl.BlockSpec((E // 2, 2 * HID, 2 * HID), lambda b, i: (0, 0, 0)),
            pl.BlockSpec((E // 2, 2 * HID, 1), lambda b, i: (0, 0, 0)),
            pl.BlockSpec((E // 2, C_OUT, 2 * HID), lambda b, i: (0, 0, 0)),
            pl.BlockSpec((C_OUT, E), lambda b, i: (0, 0)),
            pl.BlockSpec((E, R_IN), lambda b, i: (0, 0)),
            pl.BlockSpec((1, E), lambda b, i: (0, 0)),
        ],
        out_specs=pl.BlockSpec((1, C_OUT, hb, W), lambda b, i: (b, 0, i, 0)),
        out_shape=jax.ShapeDtypeStruct((B, C_OUT, H, W), jnp.float32),
    )(x, router_input, w1s, b1s, w2p, b2p, w3p, b3t, Wr, br.reshape(1, E))
    return out


# speedup vs baseline: 1.0232x; 1.0107x over previous
import math

import jax
import jax.numpy as jnp
from jax.experimental import pallas as pl

E = 8
C_IN = 96
HID = 96
C_OUT = 96
R_IN = 8

ROWS = 24
_SQRT1_2 = 1.0 / math.sqrt(2.0)


def _moe_body(x_ref, r_ref, w1_ref, b1_ref, w2_ref, b2_ref, w3_ref, b3_ref,
              wr_ref, br_ref, o_ref):
    hb, w = x_ref.shape[2], x_ref.shape[3]
    t = hb * w
    xt = x_ref[0].reshape(C_IN, t)
    rt = r_ref[0].reshape(R_IN, t)

    logits = jnp.dot(wr_ref[...], rt, preferred_element_type=jnp.float32)
    logits = logits + br_ref[...].reshape(E, 1)

    eidx = jax.lax.broadcasted_iota(jnp.int32, (E, t), 0)
    v1 = jnp.max(logits, axis=0, keepdims=True)
    idx1 = jnp.min(jnp.where(logits == v1, eidx, E), axis=0, keepdims=True)
    sel1 = eidx == idx1
    rest = jnp.where(sel1, -jnp.inf, logits)
    v2 = jnp.max(rest, axis=0, keepdims=True)
    idx2 = jnp.min(jnp.where(rest == v2, eidx, E), axis=0, keepdims=True)
    sel2 = eidx == idx2
    g1 = 1.0 / (1.0 + jnp.exp(v2 - v1))
    g2 = 1.0 - g1
    gates = jnp.where(sel1, g1, 0.0) + jnp.where(sel2, g2, 0.0)

    h = jnp.dot(w1_ref[...], xt, preferred_element_type=jnp.float32)
    h = h + b1_ref[...]
    g = h * (1.0 + jax.lax.erf(h))

    acc = jnp.dot(b3_ref[...], gates, preferred_element_type=jnp.float32)
    for p in range(E // 2):
        h2 = jnp.dot(w2_ref[p], g[192 * p:192 * (p + 1)],
                     preferred_element_type=jnp.float32)
        h2 = h2 + b2_ref[p]
        g2v = h2 * (1.0 + jax.lax.erf(h2))
        gb = jnp.concatenate(
            [jnp.broadcast_to(gates[2 * p:2 * p + 1], (HID, t)),
             jnp.broadcast_to(gates[2 * p + 1:2 * p + 2], (HID, t))], axis=0)
        acc = acc + jnp.dot(w3_ref[p], g2v * gb,
                            preferred_element_type=jnp.float32)
    o_ref[0] = acc.reshape(C_OUT, hb, w)


def kernel(x, router_input, W1, b1, W2, b2, W3, b3, Wr, br):
    B, _, H, W = x.shape
    hb = ROWS
    nt = H // hb

    w1s = (W1 * _SQRT1_2).reshape(E * HID, C_IN)
    b1s = (b1 * _SQRT1_2).reshape(E * HID, 1)
    w2h = W2 * 0.5
    w2p = jnp.zeros((E // 2, 2 * HID, 2 * HID), jnp.float32)
    w2p = w2p.at[:, :HID, :HID].set(w2h[0::2])
    w2p = w2p.at[:, HID:, HID:].set(w2h[1::2])
    b2p = (b2 * _SQRT1_2).reshape(E // 2, 2 * HID, 1)
    w3h = W3 * _SQRT1_2
    w3p = jnp.concatenate([w3h[0::2], w3h[1::2]], axis=2)
    b3t = b3.T

    grid = (B, nt)
    out = pl.pallas_call(
        _moe_body,
        grid=grid,
        in_specs=[
            pl.BlockSpec((1, C_IN, hb, W), lambda b, i: (b, 0, i, 0)),
            pl.BlockSpec((1, R_IN, hb, W), lambda b, i: (b, 0, i, 0)),
            pl.BlockSpec((E * HID, C_IN), lambda b, i: (0, 0)),
            pl.BlockSpec((E * HID, 1), lambda b, i: (0, 0)),
            pl.BlockSpec((E // 2, 2 * HID, 2 * HID), lambda b, i: (0, 0, 0)),
            pl.BlockSpec((E // 2, 2 * HID, 1), lambda b, i: (0, 0, 0)),
            pl.BlockSpec((E // 2, C_OUT, 2 * HID), lambda b, i: (0, 0, 0)),
            pl.BlockSpec((C_OUT, E), lambda b, i: (0, 0)),
            pl.BlockSpec((E, R_IN), lambda b, i: (0, 0)),
            pl.BlockSpec((1, E), lambda b, i: (0, 0)),
        ],
        out_specs=pl.BlockSpec((1, C_OUT, hb, W), lambda b, i: (b, 0, i, 0)),
        out_shape=jax.ShapeDtypeStruct((B, C_OUT, H, W), jnp.float32),
    )(x, router_input, w1s, b1s, w2p, b2p, w3p, b3t, Wr, br.reshape(1, E))
    return out
